# Initial kernel scaffold; baseline (speedup 1.0000x reference)
import jax
import jax.numpy as jnp
from jax.experimental import pallas as pl


def kernel(x, edge_index, W1, b1, W2, b2):
    raise NotImplementedError("write your pallas kernel here")


# speedup vs baseline: 7.1651x; 7.1651x over previous
import jax
import jax.numpy as jnp
from jax import lax
from jax.experimental import pallas as pl
from jax.experimental.pallas import tpu as pltpu
from jax.experimental.pallas import tpu_sc as plsc

_EPS = 1e-09
_N = 10000
_E = 320000
_D = 128

_NC = 2
_NS = 16
_NW = _NC * _NS
_EPT = _E // _NW
_CHUNK = 80
_NCHUNKS = _EPT // _CHUNK
_NPAD = 10240
_RPT = _NPAD // _NS


def _segsum_body(y_hbm, src_hbm, dst_hbm, zero_hbm, p_hbm,
                 src_v, dst_v, rows_v, acc, sem):
    c = lax.axis_index("c")
    s = lax.axis_index("s")
    wid = c * _NS + s
    pltpu.sync_copy(src_hbm.at[wid], src_v)
    pltpu.sync_copy(dst_hbm.at[wid], dst_v)
    pltpu.sync_copy(zero_hbm.at[pl.ds(s * _RPT, _RPT)],
                    acc.at[pl.ds(s * _RPT, _RPT)])
    plsc.subcore_barrier()

    def step(j, carry):
        pltpu.async_copy(y_hbm.at[src_v.at[j]], rows_v, sem).wait()
        pltpu.sync_copy(rows_v, acc.at[dst_v.at[j]], add=True)
        return carry

    lax.fori_loop(0, _NCHUNKS, step, 0)
    plsc.subcore_barrier()
    pltpu.sync_copy(acc.at[pl.ds(s * _RPT, _RPT)],
                    p_hbm.at[c, pl.ds(s * _RPT, _RPT)])


def _segsum(y, src_c, dst_c, zeros):
    mesh = plsc.VectorSubcoreMesh(core_axis_name="c", subcore_axis_name="s")
    return pl.kernel(
        _segsum_body,
        out_type=jax.ShapeDtypeStruct((_NC, _NPAD, _D), jnp.float32),
        mesh=mesh,
        scratch_types=[
            pltpu.VMEM((_NCHUNKS, _CHUNK), jnp.int32),
            pltpu.VMEM((_NCHUNKS, _CHUNK), jnp.int32),
            pltpu.VMEM((_CHUNK, _D), jnp.float32),
            pltpu.VMEM_SHARED((_NPAD, _D), jnp.float32),
            pltpu.SemaphoreType.DMA,
        ],
    )(y, src_c, dst_c, zeros)


_BLK = 1000


def _mm_body(x_ref, w_ref, o_ref):
    o_ref[...] = jnp.dot(x_ref[...], w_ref[...],
                         preferred_element_type=jnp.float32)


def _matmul(x, w):
    return pl.pallas_call(
        _mm_body,
        grid=(_N // _BLK,),
        in_specs=[pl.BlockSpec((_BLK, _D), lambda i: (i, 0)),
                  pl.BlockSpec((_D, _D), lambda i: (0, 0))],
        out_specs=pl.BlockSpec((_BLK, _D), lambda i: (i, 0)),
        out_shape=jax.ShapeDtypeStruct((_N, _D), jnp.float32),
    )(x, w)


def _fused_body(y_ref, p_ref, b_ref, w_ref, o_ref):
    h = (1.0 + _EPS) * y_ref[...] + p_ref[0] + p_ref[1] + b_ref[...]
    h = jnp.maximum(h, 0.0)
    o_ref[...] = jnp.dot(h, w_ref[...], preferred_element_type=jnp.float32)


def _fused_mm(y, p, b, w):
    return pl.pallas_call(
        _fused_body,
        grid=(_N // _BLK,),
        in_specs=[pl.BlockSpec((_BLK, _D), lambda i: (i, 0)),
                  pl.BlockSpec((_NC, _BLK, _D), lambda i: (0, i, 0)),
                  pl.BlockSpec((1, _D), lambda i: (0, 0)),
                  pl.BlockSpec((_D, _D), lambda i: (0, 0))],
        out_specs=pl.BlockSpec((_BLK, _D), lambda i: (i, 0)),
        out_shape=jax.ShapeDtypeStruct((_N, _D), jnp.float32),
    )(y, p, b, w)


def _combine_body(y_ref, p_ref, b_ref, o_ref):
    o_ref[...] = (1.0 + _EPS) * y_ref[...] + p_ref[0] + p_ref[1] + b_ref[...]


def _combine(y, p, b):
    return pl.pallas_call(
        _combine_body,
        grid=(_N // _BLK,),
        in_specs=[pl.BlockSpec((_BLK, _D), lambda i: (i, 0)),
                  pl.BlockSpec((_NC, _BLK, _D), lambda i: (0, i, 0)),
                  pl.BlockSpec((1, _D), lambda i: (0, 0))],
        out_specs=pl.BlockSpec((_BLK, _D), lambda i: (i, 0)),
        out_shape=jax.ShapeDtypeStruct((_N, _D), jnp.float32),
    )(y, p, b)


def kernel(x, edge_index, W1, b1, W2, b2):
    src = edge_index[0].astype(jnp.int32).reshape(_NW, _NCHUNKS, _CHUNK)
    dst = edge_index[1].astype(jnp.int32).reshape(_NW, _NCHUNKS, _CHUNK)
    zeros = jnp.zeros((_NPAD, _D), jnp.float32)
    b1r = b1.reshape(1, _D)
    b2r = b2.reshape(1, _D)

    y1 = _matmul(x, W1)
    p1 = _segsum(y1, src, dst, zeros)
    y2 = _fused_mm(y1, p1, b1r, W2)
    p2 = _segsum(y2, src, dst, zeros)
    return _combine(y2, p2, b2r)


# speedup vs baseline: 8.5245x; 1.1897x over previous
import jax
import jax.numpy as jnp
from jax import lax
from jax.experimental import pallas as pl
from jax.experimental.pallas import tpu as pltpu
from jax.experimental.pallas import tpu_sc as plsc

_EPS = 1e-09
_N = 10000
_E = 320000
_D = 128
_DH = _D // 2

_NC = 2
_NS = 16
_EPT = _E // _NS
_CHUNK = 80
_NCHUNKS = _EPT // _CHUNK
_RPT = _N // _NS


def _segsum_body(y_hbm, src_hbm, dst_hbm, zero_hbm, p_hbm,
                 src_v, dst_v, rows, acc, gsem):
    c = lax.axis_index("c")
    s = lax.axis_index("s")
    pltpu.sync_copy(src_hbm.at[s], src_v)
    pltpu.sync_copy(dst_hbm.at[s], dst_v)
    pltpu.sync_copy(zero_hbm.at[pl.ds(s * _RPT, _RPT)],
                    acc.at[pl.ds(s * _RPT, _RPT)])
    plsc.subcore_barrier()

    def step(j, carry):
        p = lax.rem(j, 2)
        pn = 1 - p

        @pl.when(j < _NCHUNKS)
        def _():
            pltpu.async_copy(y_hbm.at[c].at[src_v.at[j]], rows.at[p],
                             gsem.at[p])

        @pl.when(j > 0)
        def _():
            pltpu.make_async_copy(y_hbm.at[c].at[src_v.at[0]], rows.at[pn],
                                  gsem.at[pn]).wait()
            pltpu.sync_copy(rows.at[pn], acc.at[dst_v.at[j - 1]], add=True)

        return carry

    lax.fori_loop(0, _NCHUNKS + 1, step, 0)
    plsc.subcore_barrier()
    pltpu.sync_copy(acc.at[pl.ds(s * _RPT, _RPT)],
                    p_hbm.at[c, pl.ds(s * _RPT, _RPT)])


def _segsum(y_stk, src_c, dst_c, zeros):
    mesh = plsc.VectorSubcoreMesh(core_axis_name="c", subcore_axis_name="s")
    return pl.kernel(
        _segsum_body,
        out_type=jax.ShapeDtypeStruct((_NC, _N, _DH), jnp.float32),
        mesh=mesh,
        compiler_params=pltpu.CompilerParams(use_tc_tiling_on_sc=False),
        scratch_types=[
            pltpu.VMEM((_NCHUNKS, _CHUNK), jnp.int32),
            pltpu.VMEM((_NCHUNKS, _CHUNK), jnp.int32),
            pltpu.VMEM((2, _CHUNK, _DH), jnp.float32),
            pltpu.VMEM_SHARED((_N, _DH), jnp.float32),
            pltpu.SemaphoreType.DMA((2,)),
        ],
    )(y_stk, src_c, dst_c, zeros)


_BLK = 1000


def _mm_body(x_ref, w_ref, o_ref):
    r = jnp.dot(x_ref[...], w_ref[...], preferred_element_type=jnp.float32)
    o_ref[0] = r[:, :_DH]
    o_ref[1] = r[:, _DH:]


def _matmul(x, w):
    return pl.pallas_call(
        _mm_body,
        grid=(_N // _BLK,),
        in_specs=[pl.BlockSpec((_BLK, _D), lambda i: (i, 0)),
                  pl.BlockSpec((_D, _D), lambda i: (0, 0))],
        out_specs=pl.BlockSpec((_NC, _BLK, _DH), lambda i: (0, i, 0)),
        out_shape=jax.ShapeDtypeStruct((_NC, _N, _DH), jnp.float32),
    )(x, w)


def _fused_body(y_ref, p_ref, b_ref, w_ref, o_ref):
    y = jnp.concatenate([y_ref[0], y_ref[1]], axis=-1)
    a = jnp.concatenate([p_ref[0], p_ref[1]], axis=-1)
    h = (1.0 + _EPS) * y + a + b_ref[...]
    h = jnp.maximum(h, 0.0)
    r = jnp.dot(h, w_ref[...], preferred_element_type=jnp.float32)
    o_ref[0] = r[:, :_DH]
    o_ref[1] = r[:, _DH:]


def _fused_mm(y, p, b, w):
    return pl.pallas_call(
        _fused_body,
        grid=(_N // _BLK,),
        in_specs=[pl.BlockSpec((_NC, _BLK, _DH), lambda i: (0, i, 0)),
                  pl.BlockSpec((_NC, _BLK, _DH), lambda i: (0, i, 0)),
                  pl.BlockSpec((1, _D), lambda i: (0, 0)),
                  pl.BlockSpec((_D, _D), lambda i: (0, 0))],
        out_specs=pl.BlockSpec((_NC, _BLK, _DH), lambda i: (0, i, 0)),
        out_shape=jax.ShapeDtypeStruct((_NC, _N, _DH), jnp.float32),
    )(y, p, b, w)


def _combine_body(y_ref, p_ref, b_ref, o_ref):
    y = jnp.concatenate([y_ref[0], y_ref[1]], axis=-1)
    a = jnp.concatenate([p_ref[0], p_ref[1]], axis=-1)
    o_ref[...] = (1.0 + _EPS) * y + a + b_ref[...]


def _combine(y, p, b):
    return pl.pallas_call(
        _combine_body,
        grid=(_N // _BLK,),
        in_specs=[pl.BlockSpec((_NC, _BLK, _DH), lambda i: (0, i, 0)),
                  pl.BlockSpec((_NC, _BLK, _DH), lambda i: (0, i, 0)),
                  pl.BlockSpec((1, _D), lambda i: (0, 0))],
        out_specs=pl.BlockSpec((_BLK, _D), lambda i: (i, 0)),
        out_shape=jax.ShapeDtypeStruct((_N, _D), jnp.float32),
    )(y, p, b)


def kernel(x, edge_index, W1, b1, W2, b2):
    src = edge_index[0].astype(jnp.int32).reshape(_NS, _NCHUNKS, _CHUNK)
    dst = edge_index[1].astype(jnp.int32).reshape(_NS, _NCHUNKS, _CHUNK)
    zeros = jnp.zeros((_N, _DH), jnp.float32)
    b1r = b1.reshape(1, _D)
    b2r = b2.reshape(1, _D)

    y1 = _matmul(x, W1)
    p1 = _segsum(y1, src, dst, zeros)
    y2 = _fused_mm(y1, p1, b1r, W2)
    p2 = _segsum(y2, src, dst, zeros)
    return _combine(y2, p2, b2r)


# speedup vs baseline: 9.3555x; 1.0975x over previous
import jax
import jax.numpy as jnp
from jax import lax
from jax.experimental import pallas as pl
from jax.experimental.pallas import tpu as pltpu
from jax.experimental.pallas import tpu_sc as plsc

_EPS = 1e-09
_N = 10000
_E = 320000
_D = 128
_DH = _D // 2

_NC = 2
_NS = 16
_EPT = _E // _NS
_CHUNK = 80
_NCHUNKS = _EPT // _CHUNK
_RPT = _N // _NS
_NBUF = 4


def _segsum_body(y_hbm, src_hbm, dst_hbm, zero_hbm, p_hbm,
                 src_v, dst_v, rows, acc, gsem, ssem):
    c = lax.axis_index("c")
    s = lax.axis_index("s")
    pltpu.sync_copy(src_hbm.at[s], src_v)
    pltpu.sync_copy(dst_hbm.at[s], dst_v)
    pltpu.sync_copy(zero_hbm.at[pl.ds(s * _RPT, _RPT)],
                    acc.at[pl.ds(s * _RPT, _RPT)])
    plsc.subcore_barrier()

    def step(j, carry):
        p = lax.rem(j, _NBUF)
        pn = lax.rem(j + _NBUF - 1, _NBUF)

        @pl.when(j < _NCHUNKS)
        def _():
            @pl.when(j >= _NBUF)
            def _():
                pltpu.make_async_copy(rows.at[p], acc.at[dst_v.at[0]],
                                      ssem.at[p]).wait()

            pltpu.async_copy(y_hbm.at[c].at[src_v.at[j]], rows.at[p],
                             gsem.at[p])

        @pl.when(j > 0)
        def _():
            pltpu.make_async_copy(y_hbm.at[c].at[src_v.at[0]], rows.at[pn],
                                  gsem.at[pn]).wait()
            pltpu.async_copy(rows.at[pn], acc.at[dst_v.at[j - 1]],
                             ssem.at[pn], add=True)

        return carry

    lax.fori_loop(0, _NCHUNKS + 1, step, 0)

    def drain(k, carry):
        p = lax.rem(k, _NBUF)
        pltpu.make_async_copy(rows.at[p], acc.at[dst_v.at[0]],
                              ssem.at[p]).wait()
        return carry

    lax.fori_loop(_NCHUNKS - _NBUF, _NCHUNKS, drain, 0)
    plsc.subcore_barrier()
    pltpu.sync_copy(acc.at[pl.ds(s * _RPT, _RPT)],
                    p_hbm.at[c, pl.ds(s * _RPT, _RPT)])


def _segsum(y_stk, src_c, dst_c, zeros):
    mesh = plsc.VectorSubcoreMesh(core_axis_name="c", subcore_axis_name="s")
    return pl.kernel(
        _segsum_body,
        out_type=jax.ShapeDtypeStruct((_NC, _N, _DH), jnp.float32),
        mesh=mesh,
        compiler_params=pltpu.CompilerParams(use_tc_tiling_on_sc=False),
        scratch_types=[
            pltpu.VMEM((_NCHUNKS, _CHUNK), jnp.int32),
            pltpu.VMEM((_NCHUNKS, _CHUNK), jnp.int32),
            pltpu.VMEM((_NBUF, _CHUNK, _DH), jnp.float32),
            pltpu.VMEM_SHARED((_N, _DH), jnp.float32),
            pltpu.SemaphoreType.DMA((_NBUF,)),
            pltpu.SemaphoreType.DMA((_NBUF,)),
        ],
    )(y_stk, src_c, dst_c, zeros)


_BLK = 1000


def _mm_body(x_ref, w_ref, o_ref):
    r = jnp.dot(x_ref[...], w_ref[...], preferred_element_type=jnp.float32)
    o_ref[0] = r[:, :_DH]
    o_ref[1] = r[:, _DH:]


def _matmul(x, w):
    return pl.pallas_call(
        _mm_body,
        grid=(_N // _BLK,),
        in_specs=[pl.BlockSpec((_BLK, _D), lambda i: (i, 0)),
                  pl.BlockSpec((_D, _D), lambda i: (0, 0))],
        out_specs=pl.BlockSpec((_NC, _BLK, _DH), lambda i: (0, i, 0)),
        out_shape=jax.ShapeDtypeStruct((_NC, _N, _DH), jnp.float32),
    )(x, w)


def _fused_body(y_ref, p_ref, b_ref, w_ref, o_ref):
    y = jnp.concatenate([y_ref[0], y_ref[1]], axis=-1)
    a = jnp.concatenate([p_ref[0], p_ref[1]], axis=-1)
    h = (1.0 + _EPS) * y + a + b_ref[...]
    h = jnp.maximum(h, 0.0)
    r = jnp.dot(h, w_ref[...], preferred_element_type=jnp.float32)
    o_ref[0] = r[:, :_DH]
    o_ref[1] = r[:, _DH:]


def _fused_mm(y, p, b, w):
    return pl.pallas_call(
        _fused_body,
        grid=(_N // _BLK,),
        in_specs=[pl.BlockSpec((_NC, _BLK, _DH), lambda i: (0, i, 0)),
                  pl.BlockSpec((_NC, _BLK, _DH), lambda i: (0, i, 0)),
                  pl.BlockSpec((1, _D), lambda i: (0, 0)),
                  pl.BlockSpec((_D, _D), lambda i: (0, 0))],
        out_specs=pl.BlockSpec((_NC, _BLK, _DH), lambda i: (0, i, 0)),
        out_shape=jax.ShapeDtypeStruct((_NC, _N, _DH), jnp.float32),
    )(y, p, b, w)


def _combine_body(y_ref, p_ref, b_ref, o_ref):
    y = jnp.concatenate([y_ref[0], y_ref[1]], axis=-1)
    a = jnp.concatenate([p_ref[0], p_ref[1]], axis=-1)
    o_ref[...] = (1.0 + _EPS) * y + a + b_ref[...]


def _combine(y, p, b):
    return pl.pallas_call(
        _combine_body,
        grid=(_N // _BLK,),
        in_specs=[pl.BlockSpec((_NC, _BLK, _DH), lambda i: (0, i, 0)),
                  pl.BlockSpec((_NC, _BLK, _DH), lambda i: (0, i, 0)),
                  pl.BlockSpec((1, _D), lambda i: (0, 0))],
        out_specs=pl.BlockSpec((_BLK, _D), lambda i: (i, 0)),
        out_shape=jax.ShapeDtypeStruct((_N, _D), jnp.float32),
    )(y, p, b)


def kernel(x, edge_index, W1, b1, W2, b2):
    src = edge_index[0].astype(jnp.int32).reshape(_NS, _NCHUNKS, _CHUNK)
    dst = edge_index[1].astype(jnp.int32).reshape(_NS, _NCHUNKS, _CHUNK)
    zeros = jnp.zeros((_N, _DH), jnp.float32)
    b1r = b1.reshape(1, _D)
    b2r = b2.reshape(1, _D)

    y1 = _matmul(x, W1)
    p1 = _segsum(y1, src, dst, zeros)
    y2 = _fused_mm(y1, p1, b1r, W2)
    p2 = _segsum(y2, src, dst, zeros)
    return _combine(y2, p2, b2r)
